# Initial kernel scaffold; baseline (speedup 1.0000x reference)
import jax
import jax.numpy as jnp
from jax.experimental import pallas as pl


def kernel(query_x, query_attention_mask, product_x, edge_index_q2p, edge_weight_q2p, edge_index_p2q, edge_weight_p2q, token_table, Wq, bq, Wp, bp, W_self_q, W_nbr_q, b_gq, W_self_p, W_nbr_p, b_gp):
    raise NotImplementedError("write your pallas kernel here")


# speedup vs baseline: 2.8692x; 2.8692x over previous
import functools

import jax
import jax.numpy as jnp
from jax import lax
from jax.experimental import pallas as pl
from jax.experimental.pallas import tpu as pltpu
from jax.experimental.pallas import tpu_sc as plsc

Nq, Np, L, V, D, E = 10000, 10000, 32, 30000, 128, 320000

NW = 32
QCHUNK = 320
NQP = NW * QCHUNK
GROUP = 4
NGROUPS = QCHUNK // GROUP
DUMP = 4

EDGE_CHUNK = 128
NCHUNKS = -(-E // (16 * EDGE_CHUNK))
EP = NCHUNKS * 16 * EDGE_CHUNK
E_PER_TILE = EP // 16

_mesh = plsc.VectorSubcoreMesh(core_axis_name="c", subcore_axis_name="s")


@functools.partial(
    pl.kernel,
    out_type=jax.ShapeDtypeStruct((NQP, D), jnp.float32),
    mesh=_mesh,
    scratch_types=[
        pltpu.VMEM((EDGE_CHUNK,), jnp.int32),
        pltpu.VMEM((EDGE_CHUNK,), jnp.int32),
        pltpu.VMEM((EDGE_CHUNK,), jnp.int32),
        pltpu.VMEM((EDGE_CHUNK, D), jnp.float32),
        pltpu.VMEM((8, D), jnp.float32),
        pltpu.VMEM((QCHUNK, D), jnp.float32),
        pltpu.VMEM_SHARED((16 * 8, D), jnp.float32),
        pltpu.SemaphoreType.DMA,
    ],
)
def _pool_kernel(qx_hbm, qm_hbm, table_hbm, out_hbm,
                 idx_v, msk_v, dst_v, rows_v, acc_v, pooled_v, acc_sh, sem):
    c = lax.axis_index("c")
    s = lax.axis_index("s")
    w = s * 2 + c
    qbase = w * QCHUNK
    slot0 = s * 8

    zeros = jnp.zeros((16,), jnp.float32)
    for slot in range(8):
        for j in range(D // 16):
            acc_v[slot, pl.ds(j * 16, 16)] = zeros

    def group_body(g, carry):
        tbase = (qbase + g * GROUP) * L
        pltpu.sync_copy(qx_hbm.at[pl.ds(tbase, GROUP * L)], idx_v)
        pltpu.sync_copy(qm_hbm.at[pl.ds(tbase, GROUP * L)], msk_v)

        pltpu.sync_copy(acc_v.at[pl.ds(0, DUMP + 1)],
                        acc_sh.at[pl.ds(slot0, DUMP + 1)])

        one = jnp.full((16,), 1, jnp.int32)
        zero16 = jnp.full((16,), 0, jnp.int32)
        lanes = jnp.arange(16, dtype=jnp.int32)
        counts = []
        for q in range(GROUP):
            cnt = None
            for half in range(L // 16):
                j = q * (L // 16) + half
                m = msk_v[pl.ds(j * 16, 16)]
                valid = m != 0
                dst_v[pl.ds(j * 16, 16)] = jnp.where(
                    valid, jnp.full((16,), slot0 + q, jnp.int32),
                    jnp.full((16,), slot0 + DUMP, jnp.int32))
                part = jnp.where(valid, one, zero16)
                cnt = part if cnt is None else cnt + part
            for kk in (1, 2, 4, 8):
                cnt = cnt + jnp.take(cnt, (lanes + kk) % 16)
            counts.append(cnt)

        pltpu.async_copy(table_hbm.at[idx_v], rows_v, sem).wait()
        pltpu.sync_copy(rows_v, acc_sh.at[dst_v], add=True)
        pltpu.sync_copy(acc_sh.at[pl.ds(slot0, GROUP)],
                        rows_v.at[pl.ds(0, GROUP)])

        for q in range(GROUP):
            dvec = jnp.maximum(counts[q].astype(jnp.float32), 1.0)
            row = g * GROUP + q
            for j in range(D // 16):
                pooled_v[row, pl.ds(j * 16, 16)] = (
                    rows_v[q, pl.ds(j * 16, 16)] / dvec)
        return carry

    lax.fori_loop(0, NGROUPS, group_body, 0)
    pltpu.sync_copy(pooled_v, out_hbm.at[pl.ds(qbase, QCHUNK)])


@functools.partial(
    pl.kernel,
    out_type=jax.ShapeDtypeStruct((Nq + Np, D), jnp.float32),
    mesh=_mesh,
    scratch_types=[
        pltpu.VMEM((EDGE_CHUNK,), jnp.int32),
        pltpu.VMEM((EDGE_CHUNK,), jnp.int32),
        pltpu.VMEM((EDGE_CHUNK,), jnp.float32),
        pltpu.VMEM((EDGE_CHUNK, D), jnp.float32),
        pltpu.VMEM_SHARED((10240, D), jnp.float32),
        pltpu.SemaphoreType.DMA,
    ],
)
def _edge_kernel(h_hbm, src_hbm, dst_hbm, w_hbm, out_hbm,
                 src_v, dst_v, w_v, rows_v, agg_sh, sem):
    c = lax.axis_index("c")
    s = lax.axis_index("s")

    zeros = jnp.zeros((16,), jnp.float32)

    def zero_body(r, carry):
        for j in range(D // 16):
            rows_v[r, pl.ds(j * 16, 16)] = zeros
        return carry

    lax.fori_loop(0, EDGE_CHUNK, zero_body, 0)
    zbase = s * 640
    for k in range(5):
        pltpu.sync_copy(rows_v,
                        agg_sh.at[pl.ds(zbase + k * 128, 128)])
    plsc.subcore_barrier()

    ebase = c * EP + s * E_PER_TILE

    def chunk_body(t, carry):
        base = ebase + t * EDGE_CHUNK
        pltpu.sync_copy(src_hbm.at[pl.ds(base, EDGE_CHUNK)], src_v)
        pltpu.sync_copy(dst_hbm.at[pl.ds(base, EDGE_CHUNK)], dst_v)
        pltpu.sync_copy(w_hbm.at[pl.ds(base, EDGE_CHUNK)], w_v)
        pltpu.async_copy(h_hbm.at[src_v], rows_v, sem).wait()
        for grp in range(EDGE_CHUNK // 16):
            w16 = w_v[pl.ds(grp * 16, 16)]
            for lane in range(16):
                e = grp * 16 + lane
                wspl = jnp.take(w16, jnp.full((16,), lane, jnp.int32))
                for j in range(D // 16):
                    rows_v[e, pl.ds(j * 16, 16)] = (
                        rows_v[e, pl.ds(j * 16, 16)] * wspl)
        pltpu.sync_copy(rows_v, agg_sh.at[dst_v], add=True)
        return carry

    lax.fori_loop(0, NCHUNKS, chunk_body, 0)
    plsc.subcore_barrier()

    obase = (1 - c) * Nq + zbase

    @pl.when(s < 15)
    def _():
        for k in range(5):
            pltpu.sync_copy(agg_sh.at[pl.ds(zbase + k * 128, 128)],
                            out_hbm.at[pl.ds(obase + k * 128, 128)])

    @pl.when(s == 15)
    def _():
        for k, sz in ((0, 128), (1, 128), (2, 128), (3, 16)):
            pltpu.sync_copy(agg_sh.at[pl.ds(zbase + k * 128, sz)],
                            out_hbm.at[pl.ds(obase + k * 128, sz)])


def _dense_body(x_ref, w_ref, b_ref, o_ref):
    o_ref[...] = jnp.dot(x_ref[...], w_ref[0],
                         preferred_element_type=jnp.float32) + b_ref[0]


def _dense(x, w_st, b_st, rows_per_type, block):
    n = x.shape[0]
    grid = n // block
    per_type = rows_per_type // block
    return pl.pallas_call(
        _dense_body,
        grid=(grid,),
        in_specs=[
            pl.BlockSpec((block, D), lambda i: (i, 0)),
            pl.BlockSpec((1, D, D), lambda i: (i // per_type, 0, 0)),
            pl.BlockSpec((1, 1, D), lambda i: (i // per_type, 0, 0)),
        ],
        out_specs=pl.BlockSpec((block, D), lambda i: (i, 0)),
        out_shape=jax.ShapeDtypeStruct((n, D), jnp.float32),
    )(x, w_st, b_st)


def _gnn_body(h_ref, a_ref, ws_ref, wn_ref, b_ref, o_ref):
    acc = jnp.dot(h_ref[...], ws_ref[0], preferred_element_type=jnp.float32)
    acc += jnp.dot(a_ref[...], wn_ref[0], preferred_element_type=jnp.float32)
    o_ref[...] = jnp.maximum(acc + b_ref[0], 0.0)


def _gnn_out(h, agg, ws_st, wn_st, b_st, block):
    n = h.shape[0]
    grid = n // block
    per_type = (n // 2) // block
    return pl.pallas_call(
        _gnn_body,
        grid=(grid,),
        in_specs=[
            pl.BlockSpec((block, D), lambda i: (i, 0)),
            pl.BlockSpec((block, D), lambda i: (i, 0)),
            pl.BlockSpec((1, D, D), lambda i: (i // per_type, 0, 0)),
            pl.BlockSpec((1, D, D), lambda i: (i // per_type, 0, 0)),
            pl.BlockSpec((1, 1, D), lambda i: (i // per_type, 0, 0)),
        ],
        out_specs=pl.BlockSpec((block, D), lambda i: (i, 0)),
        out_shape=jax.ShapeDtypeStruct((n, D), jnp.float32),
    )(h, agg, ws_st, wn_st, b_st)


def kernel(query_x, query_attention_mask, product_x,
           edge_index_q2p, edge_weight_q2p,
           edge_index_p2q, edge_weight_p2q,
           token_table, Wq, bq, Wp, bp,
           W_self_q, W_nbr_q, b_gq,
           W_self_p, W_nbr_p, b_gp):
    qx = jnp.pad(query_x.astype(jnp.int32), ((0, NQP - Nq), (0, 0)))
    qm = jnp.pad(query_attention_mask.astype(jnp.int32),
                 ((0, NQP - Nq), (0, 0)))
    pooled = _pool_kernel(qx.reshape(-1), qm.reshape(-1), token_table)

    xcat = jnp.concatenate([pooled[:Nq], product_x], axis=0)
    w_st = jnp.stack([Wq, Wp])
    b_st = jnp.stack([bq, bp])[:, None, :]
    hcat = _dense(xcat, w_st, b_st, Nq, 1000)

    pad = EP - E
    src = jnp.concatenate([
        jnp.pad(edge_index_q2p[0].astype(jnp.int32), (0, pad)),
        jnp.pad(edge_index_p2q[0].astype(jnp.int32), (0, pad)) + Nq])
    dst = jnp.concatenate([
        jnp.pad(edge_index_q2p[1].astype(jnp.int32), (0, pad)),
        jnp.pad(edge_index_p2q[1].astype(jnp.int32), (0, pad))])
    ew = jnp.concatenate([jnp.pad(edge_weight_q2p, (0, pad)),
                          jnp.pad(edge_weight_p2q, (0, pad))])
    agg = _edge_kernel(hcat, src, dst, ew)

    ws_st = jnp.stack([W_self_q, W_self_p])
    wn_st = jnp.stack([W_nbr_q, W_nbr_p])
    bg_st = jnp.stack([b_gq, b_gp])[:, None, :]
    return _gnn_out(hcat, agg, ws_st, wn_st, bg_st, 1000)
